# initial kernel scaffold (unmeasured)
import jax
import jax.numpy as jnp
from jax import lax
from jax.experimental import pallas as pl
from jax.experimental.pallas import tpu as pltpu

N_DEV = 8
M, N = 4096, 2048
CHUNK = M // N_DEV

RING = (0, 1, 2, 3, 7, 6, 5, 4)
RANK_OF = (0, 1, 2, 3, 7, 6, 5, 4)


def kernel(x, w_mat, scale_x, scale_w):
    def body(x_ref, w_ref, sx_ref, sw_ref, out_ref,
             comm_ref, rs_send_sems, rs_recv_sems, ag_send_sems, ag_recv_sems):
        my_pos = lax.axis_index("i")
        ring = jnp.array(RING, dtype=jnp.int32)
        rank = jnp.array(RANK_OF, dtype=jnp.int32)[my_pos]
        right = ring[(rank + 1) % N_DEV]
        left = ring[(rank - 1) % N_DEV]

        barrier_sem = pltpu.get_barrier_semaphore()
        for nbr in (left, right):
            pl.semaphore_signal(
                barrier_sem, inc=1,
                device_id=(nbr,), device_id_type=pl.DeviceIdType.MESH,
            )
        pl.semaphore_wait(barrier_sem, 2)

        s = sx_ref[0] * sw_ref[0]
        out_ref[...] = (
            jnp.dot(x_ref[...], w_ref[...], preferred_element_type=jnp.float32)
            * s
        )

        for h in range(N_DEV - 1):
            send_chunk = (rank - h) % N_DEV
            rdma = pltpu.make_async_remote_copy(
                src_ref=out_ref.at[pl.ds(send_chunk * CHUNK, CHUNK), :],
                dst_ref=comm_ref.at[h],
                send_sem=rs_send_sems.at[h],
                recv_sem=rs_recv_sems.at[h],
                device_id=(right,),
                device_id_type=pl.DeviceIdType.MESH,
            )
            rdma.start()
            rdma.wait()
            recv_chunk = (rank - h - 1) % N_DEV
            rows = pl.ds(recv_chunk * CHUNK, CHUNK)
            out_ref[rows, :] = out_ref[rows, :] + comm_ref[h]

        for h in range(N_DEV - 1):
            send_chunk = (rank + 1 - h) % N_DEV
            rows = pl.ds(send_chunk * CHUNK, CHUNK)
            rdma = pltpu.make_async_remote_copy(
                src_ref=out_ref.at[rows, :],
                dst_ref=out_ref.at[rows, :],
                send_sem=ag_send_sems.at[h],
                recv_sem=ag_recv_sems.at[h],
                device_id=(right,),
                device_id_type=pl.DeviceIdType.MESH,
            )
            rdma.start()
            rdma.wait()

    return pl.pallas_call(
        body,
        out_shape=jax.ShapeDtypeStruct((M, N), jnp.float32),
        in_specs=[
            pl.BlockSpec(memory_space=pltpu.VMEM),
            pl.BlockSpec(memory_space=pltpu.VMEM),
            pl.BlockSpec(memory_space=pltpu.VMEM),
            pl.BlockSpec(memory_space=pltpu.VMEM),
        ],
        out_specs=pl.BlockSpec(memory_space=pltpu.VMEM),
        scratch_shapes=[
            pltpu.VMEM((N_DEV - 1, CHUNK, N), jnp.float32),
            pltpu.SemaphoreType.DMA((N_DEV - 1,)),
            pltpu.SemaphoreType.DMA((N_DEV - 1,)),
            pltpu.SemaphoreType.DMA((N_DEV - 1,)),
            pltpu.SemaphoreType.DMA((N_DEV - 1,)),
        ],
        compiler_params=pltpu.CompilerParams(collective_id=0),
    )(x, w_mat, scale_x, scale_w)


# baseline (device time: 707026 ns/iter reference)
import jax
import jax.numpy as jnp
from jax import lax
from jax.experimental import pallas as pl
from jax.experimental.pallas import tpu as pltpu

N_DEV = 8
M, N = 4096, 2048
CHUNK = M // N_DEV


def kernel(x, w_mat, scale_x, scale_w):
    def body(x_ref, w_ref, sx_ref, sw_ref, out_ref,
             comm_ref, rs_send_sems, rs_recv_sems, credit_sems,
             ag_send_sems, ag_recv_sems):
        my_pos = lax.axis_index("i")

        def ring_map(p):
            return jnp.where(p < 4, p, 11 - p)

        rank = ring_map(my_pos)
        right = ring_map((rank + 1) % N_DEV)
        left = ring_map((rank - 1) % N_DEV)

        barrier_sem = pltpu.get_barrier_semaphore()
        for nbr in (left, right):
            pl.semaphore_signal(
                barrier_sem, inc=1,
                device_id=(nbr,), device_id_type=pl.DeviceIdType.MESH,
            )
        pl.semaphore_wait(barrier_sem, 2)

        s = (sx_ref[0] * sw_ref[0]).astype(jnp.float32)
        w_bf = w_ref[...].astype(jnp.bfloat16)
        for c in range(N_DEV):
            rows = pl.ds(c * CHUNK, CHUNK)
            out_ref[rows, :] = (
                jnp.dot(x_ref[rows, :].astype(jnp.bfloat16), w_bf,
                        preferred_element_type=jnp.float32)
                * s
            )

        for h in range(N_DEV - 1):
            slot = h % 2
            if h >= 2:
                pl.semaphore_wait(credit_sems.at[slot], 1)
            send_chunk = (rank - h) % N_DEV
            rdma = pltpu.make_async_remote_copy(
                src_ref=out_ref.at[pl.ds(send_chunk * CHUNK, CHUNK), :],
                dst_ref=comm_ref.at[slot],
                send_sem=rs_send_sems.at[slot],
                recv_sem=rs_recv_sems.at[slot],
                device_id=(right,),
                device_id_type=pl.DeviceIdType.MESH,
            )
            rdma.start()
            rdma.wait()
            recv_chunk = (rank - h - 1) % N_DEV
            rows = pl.ds(recv_chunk * CHUNK, CHUNK)
            out_ref[rows, :] = out_ref[rows, :] + comm_ref[slot]
            if h < N_DEV - 3:
                pl.semaphore_signal(
                    credit_sems.at[slot], inc=1,
                    device_id=(left,), device_id_type=pl.DeviceIdType.MESH,
                )

        for h in range(N_DEV - 1):
            send_chunk = (rank + 1 - h) % N_DEV
            rows = pl.ds(send_chunk * CHUNK, CHUNK)
            rdma = pltpu.make_async_remote_copy(
                src_ref=out_ref.at[rows, :],
                dst_ref=out_ref.at[rows, :],
                send_sem=ag_send_sems.at[h],
                recv_sem=ag_recv_sems.at[h],
                device_id=(right,),
                device_id_type=pl.DeviceIdType.MESH,
            )
            rdma.start()
            rdma.wait()

    return pl.pallas_call(
        body,
        out_shape=jax.ShapeDtypeStruct((M, N), jnp.float32),
        in_specs=[
            pl.BlockSpec(memory_space=pltpu.VMEM),
            pl.BlockSpec(memory_space=pltpu.VMEM),
            pl.BlockSpec(memory_space=pltpu.VMEM),
            pl.BlockSpec(memory_space=pltpu.VMEM),
        ],
        out_specs=pl.BlockSpec(memory_space=pltpu.VMEM),
        scratch_shapes=[
            pltpu.VMEM((2, CHUNK, N), jnp.float32),
            pltpu.SemaphoreType.DMA((2,)),
            pltpu.SemaphoreType.DMA((2,)),
            pltpu.SemaphoreType.REGULAR((2,)),
            pltpu.SemaphoreType.DMA((N_DEV - 1,)),
            pltpu.SemaphoreType.DMA((N_DEV - 1,)),
        ],
        compiler_params=pltpu.CompilerParams(
            collective_id=0,
            vmem_limit_bytes=58 * 1024 * 1024,
        ),
    )(x, w_mat, scale_x, scale_w)


# device time: 242009 ns/iter; 2.9215x vs baseline; 2.9215x over previous
import jax
import jax.numpy as jnp
from jax import lax
from jax.experimental import pallas as pl
from jax.experimental.pallas import tpu as pltpu

N_DEV = 8
M, N = 4096, 2048
CHUNK = M // N_DEV
HALF = N // 2


def kernel(x, w_mat, scale_x, scale_w):
    def body(x_ref, w_ref, sx_ref, sw_ref, out_ref,
             rs_buf, rs_stage, rs_send_sems, rs_recv_sems, rs_credits,
             ag_buf, ag_send_sems, ag_recv_sems, ag_credits):
        my_pos = lax.axis_index("i")

        def ring_map(p):
            return jnp.where(p < 4, p, 11 - p)

        rank_cw = ring_map(my_pos)
        right = ring_map((rank_cw + 1) % N_DEV)
        left = ring_map((rank_cw - 1) % N_DEV)
        rank_ccw = (N_DEV - rank_cw) % N_DEV
        ranks = (rank_cw, rank_ccw)
        nxt = (right, left)
        prv = (left, right)
        col0 = (0, HALF)

        barrier_sem = pltpu.get_barrier_semaphore()
        for nbr in (left, right):
            pl.semaphore_signal(
                barrier_sem, inc=1,
                device_id=(nbr,), device_id_type=pl.DeviceIdType.MESH,
            )
        pl.semaphore_wait(barrier_sem, 2)

        s = (sx_ref[0] * sw_ref[0]).astype(jnp.float32)
        w_bf = w_ref[...].astype(jnp.bfloat16)
        for c in range(N_DEV):
            rows = pl.ds(c * CHUNK, CHUNK)
            out_ref[rows, :] = (
                jnp.dot(x_ref[rows, :].astype(jnp.bfloat16), w_bf,
                        preferred_element_type=jnp.float32)
                * s
            )

        for d in range(2):
            rows = pl.ds(ranks[d] * CHUNK, CHUNK)
            rs_stage[d] = out_ref[rows, pl.ds(col0[d], HALF)].astype(jnp.bfloat16)

        for h in range(N_DEV - 1):
            slot = h % 2
            rdmas = []
            for d in range(2):
                if h >= 2:
                    pl.semaphore_wait(rs_credits.at[d, slot], 1)
                rdma = pltpu.make_async_remote_copy(
                    src_ref=rs_stage.at[d],
                    dst_ref=rs_buf.at[d, slot],
                    send_sem=rs_send_sems.at[d, slot],
                    recv_sem=rs_recv_sems.at[d, slot],
                    device_id=(nxt[d],),
                    device_id_type=pl.DeviceIdType.MESH,
                )
                rdma.start()
                rdmas.append(rdma)
            for d in range(2):
                rdmas[d].wait()
                recv_chunk = (ranks[d] - h - 1) % N_DEV
                rows = pl.ds(recv_chunk * CHUNK, CHUNK)
                cols = pl.ds(col0[d], HALF)
                acc = out_ref[rows, cols] + rs_buf[d, slot].astype(jnp.float32)
                out_ref[rows, cols] = acc
                rs_stage[d] = acc.astype(jnp.bfloat16)
                if h < N_DEV - 3:
                    pl.semaphore_signal(
                        rs_credits.at[d, slot], inc=1,
                        device_id=(prv[d],),
                        device_id_type=pl.DeviceIdType.MESH,
                    )

        for h in range(N_DEV - 1):
            slot = h % 2
            rdmas = []
            for d in range(2):
                if h >= 2:
                    pl.semaphore_wait(ag_credits.at[d, slot], 1)
                src = rs_stage.at[d] if h == 0 else ag_buf.at[d, (h - 1) % 2]
                rdma = pltpu.make_async_remote_copy(
                    src_ref=src,
                    dst_ref=ag_buf.at[d, slot],
                    send_sem=ag_send_sems.at[d, slot],
                    recv_sem=ag_recv_sems.at[d, slot],
                    device_id=(nxt[d],),
                    device_id_type=pl.DeviceIdType.MESH,
                )
                rdma.start()
                rdmas.append(rdma)
            for d in range(2):
                rdmas[d].wait()
                if 1 <= h <= 5:
                    pl.semaphore_signal(
                        ag_credits.at[d, (h - 1) % 2], inc=1,
                        device_id=(prv[d],),
                        device_id_type=pl.DeviceIdType.MESH,
                    )
                recv_chunk = (ranks[d] - h) % N_DEV
                rows = pl.ds(recv_chunk * CHUNK, CHUNK)
                out_ref[rows, pl.ds(col0[d], HALF)] = (
                    ag_buf[d, slot].astype(jnp.float32)
                )

    return pl.pallas_call(
        body,
        out_shape=jax.ShapeDtypeStruct((M, N), jnp.float32),
        in_specs=[
            pl.BlockSpec(memory_space=pltpu.VMEM),
            pl.BlockSpec(memory_space=pltpu.VMEM),
            pl.BlockSpec(memory_space=pltpu.VMEM),
            pl.BlockSpec(memory_space=pltpu.VMEM),
        ],
        out_specs=pl.BlockSpec(memory_space=pltpu.VMEM),
        scratch_shapes=[
            pltpu.VMEM((2, 2, CHUNK, HALF), jnp.bfloat16),
            pltpu.VMEM((2, CHUNK, HALF), jnp.bfloat16),
            pltpu.SemaphoreType.DMA((2, 2)),
            pltpu.SemaphoreType.DMA((2, 2)),
            pltpu.SemaphoreType.REGULAR((2, 2)),
            pltpu.VMEM((2, 2, CHUNK, HALF), jnp.bfloat16),
            pltpu.SemaphoreType.DMA((2, 2)),
            pltpu.SemaphoreType.DMA((2, 2)),
            pltpu.SemaphoreType.REGULAR((2, 2)),
        ],
        compiler_params=pltpu.CompilerParams(
            collective_id=0,
            vmem_limit_bytes=58 * 1024 * 1024,
        ),
    )(x, w_mat, scale_x, scale_w)


# device time: 211728 ns/iter; 3.3393x vs baseline; 1.1430x over previous
import jax
import jax.numpy as jnp
from jax import lax
from jax.experimental import pallas as pl
from jax.experimental.pallas import tpu as pltpu

N_DEV = 8
M, N = 4096, 2048
CHUNK = M // N_DEV
LANES = 4
LCOL = N // LANES
LANE_DIR = (0, 1, 0, 1)
LANE_COL0 = (0, 1024, 512, 1536)


def kernel(x, w_mat, scale_x, scale_w):
    def body(x_ref, w_ref, sx_ref, sw_ref, out_ref,
             stage, rs_buf, rs_send_sems, rs_recv_sems, rs_credits,
             ag_buf, ag_send_sems, ag_recv_sems, ag_credits):
        my_pos = lax.axis_index("i")

        def ring_map(p):
            return jnp.where(p < 4, p, 11 - p)

        rank_cw = ring_map(my_pos)
        right = ring_map((rank_cw + 1) % N_DEV)
        left = ring_map((rank_cw - 1) % N_DEV)
        rank_ccw = (N_DEV - rank_cw) % N_DEV
        ranks = (rank_cw, rank_ccw)
        nxt = (right, left)
        prv = (left, right)

        def lane_rank(l):
            return ranks[LANE_DIR[l]]

        def lane_nxt(l):
            return nxt[LANE_DIR[l]]

        def lane_prv(l):
            return prv[LANE_DIR[l]]

        def cols(l):
            return pl.ds(LANE_COL0[l], LCOL)

        barrier_sem = pltpu.get_barrier_semaphore()
        for nbr in (left, right):
            pl.semaphore_signal(
                barrier_sem, inc=1,
                device_id=(nbr,), device_id_type=pl.DeviceIdType.MESH,
            )
        pl.semaphore_wait(barrier_sem, 2)

        s = (sx_ref[0] * sw_ref[0]).astype(jnp.float32)
        w_bf = w_ref[...].astype(jnp.bfloat16)
        for c in range(N_DEV):
            rows = pl.ds(c * CHUNK, CHUNK)
            out_ref[rows, :] = (
                jnp.dot(x_ref[rows, :].astype(jnp.bfloat16), w_bf,
                        preferred_element_type=jnp.float32)
                * s
            )

        def rs_send(l, h):
            rdma = pltpu.make_async_remote_copy(
                src_ref=stage.at[l],
                dst_ref=rs_buf.at[l, h % 2],
                send_sem=rs_send_sems.at[l, h % 2],
                recv_sem=rs_recv_sems.at[l, h % 2],
                device_id=(lane_nxt(l),),
                device_id_type=pl.DeviceIdType.MESH,
            )
            rdma.start()
            return rdma

        def ag_send(l, h):
            src = stage.at[l] if h == 0 else ag_buf.at[l, (h - 1) % 2]
            rdma = pltpu.make_async_remote_copy(
                src_ref=src,
                dst_ref=ag_buf.at[l, h % 2],
                send_sem=ag_send_sems.at[l, h % 2],
                recv_sem=ag_recv_sems.at[l, h % 2],
                device_id=(lane_nxt(l),),
                device_id_type=pl.DeviceIdType.MESH,
            )
            rdma.start()
            return rdma

        inflight = [None] * LANES
        for l in range(LANES):
            rows = pl.ds(lane_rank(l) * CHUNK, CHUNK)
            stage[l] = out_ref[rows, cols(l)].astype(jnp.bfloat16)
            inflight[l] = rs_send(l, 0)

        for h in range(N_DEV - 1):
            slot = h % 2
            for l in range(LANES):
                inflight[l].wait()
                recv_chunk = (lane_rank(l) - h - 1) % N_DEV
                rows = pl.ds(recv_chunk * CHUNK, CHUNK)
                acc = out_ref[rows, cols(l)] + rs_buf[l, slot].astype(jnp.float32)
                stage[l] = acc.astype(jnp.bfloat16)
                if h <= 4:
                    pl.semaphore_signal(
                        rs_credits.at[l, slot], inc=1,
                        device_id=(lane_prv(l),),
                        device_id_type=pl.DeviceIdType.MESH,
                    )
                if h < N_DEV - 2:
                    if h + 1 >= 2:
                        pl.semaphore_wait(rs_credits.at[l, (h + 1) % 2], 1)
                    inflight[l] = rs_send(l, h + 1)
                else:
                    inflight[l] = ag_send(l, 0)
                out_ref[rows, cols(l)] = acc

        for h in range(N_DEV - 1):
            slot = h % 2
            for l in range(LANES):
                inflight[l].wait()
                if 1 <= h <= 5:
                    pl.semaphore_signal(
                        ag_credits.at[l, (h - 1) % 2], inc=1,
                        device_id=(lane_prv(l),),
                        device_id_type=pl.DeviceIdType.MESH,
                    )
                if h < N_DEV - 2:
                    if h + 1 >= 2:
                        pl.semaphore_wait(ag_credits.at[l, (h + 1) % 2], 1)
                    inflight[l] = ag_send(l, h + 1)
                recv_chunk = (lane_rank(l) - h) % N_DEV
                rows = pl.ds(recv_chunk * CHUNK, CHUNK)
                out_ref[rows, cols(l)] = ag_buf[l, slot].astype(jnp.float32)

    return pl.pallas_call(
        body,
        out_shape=jax.ShapeDtypeStruct((M, N), jnp.float32),
        in_specs=[
            pl.BlockSpec(memory_space=pltpu.VMEM),
            pl.BlockSpec(memory_space=pltpu.VMEM),
            pl.BlockSpec(memory_space=pltpu.VMEM),
            pl.BlockSpec(memory_space=pltpu.VMEM),
        ],
        out_specs=pl.BlockSpec(memory_space=pltpu.VMEM),
        scratch_shapes=[
            pltpu.VMEM((LANES, CHUNK, LCOL), jnp.bfloat16),
            pltpu.VMEM((LANES, 2, CHUNK, LCOL), jnp.bfloat16),
            pltpu.SemaphoreType.DMA((LANES, 2)),
            pltpu.SemaphoreType.DMA((LANES, 2)),
            pltpu.SemaphoreType.REGULAR((LANES, 2)),
            pltpu.VMEM((LANES, 2, CHUNK, LCOL), jnp.bfloat16),
            pltpu.SemaphoreType.DMA((LANES, 2)),
            pltpu.SemaphoreType.DMA((LANES, 2)),
            pltpu.SemaphoreType.REGULAR((LANES, 2)),
        ],
        compiler_params=pltpu.CompilerParams(
            collective_id=0,
            vmem_limit_bytes=58 * 1024 * 1024,
        ),
    )(x, w_mat, scale_x, scale_w)


# device time: 211619 ns/iter; 3.3410x vs baseline; 1.0005x over previous
import jax
import jax.numpy as jnp
from jax import lax
from jax.experimental import pallas as pl
from jax.experimental.pallas import tpu as pltpu

N_DEV = 8
M, N = 4096, 2048
CHUNK = M // N_DEV
LANES = 4
LCOL = N // LANES
LANE_DIR = (0, 1, 0, 1)
LANE_COL0 = (0, 1024, 512, 1536)


def kernel(x, w_mat, scale_x, scale_w):
    def body(x_ref, w_ref, sx_ref, sw_ref, out_ref,
             stage, rs_buf, rs_send_sems, rs_recv_sems, rs_credits,
             ag_buf, ag_send_sems, ag_recv_sems, ag_credits):
        my_pos = lax.axis_index("i")

        def ring_map(p):
            return jnp.where(p < 4, p, 11 - p)

        rank_cw = ring_map(my_pos)
        right = ring_map((rank_cw + 1) % N_DEV)
        left = ring_map((rank_cw - 1) % N_DEV)
        rank_ccw = (N_DEV - rank_cw) % N_DEV
        ranks = (rank_cw, rank_ccw)
        nxt = (right, left)
        prv = (left, right)

        def lane_rank(l):
            return ranks[LANE_DIR[l]]

        def lane_nxt(l):
            return nxt[LANE_DIR[l]]

        def lane_prv(l):
            return prv[LANE_DIR[l]]

        def cols(l):
            return pl.ds(LANE_COL0[l], LCOL)

        barrier_sem = pltpu.get_barrier_semaphore()
        for nbr in (left, right):
            pl.semaphore_signal(
                barrier_sem, inc=1,
                device_id=(nbr,), device_id_type=pl.DeviceIdType.MESH,
            )
        pl.semaphore_wait(barrier_sem, 2)

        s = (sx_ref[0] * sw_ref[0]).astype(jnp.float32)
        w_bf = w_ref[...].astype(jnp.bfloat16)
        for c in range(N_DEV):
            rows = pl.ds(c * CHUNK, CHUNK)
            out_ref[rows, :] = (
                jnp.dot(x_ref[rows, :].astype(jnp.bfloat16), w_bf,
                        preferred_element_type=jnp.float32)
                * s
            )

        def rs_send(l, h):
            rdma = pltpu.make_async_remote_copy(
                src_ref=stage.at[l],
                dst_ref=rs_buf.at[l, h % 2],
                send_sem=rs_send_sems.at[l, h % 2],
                recv_sem=rs_recv_sems.at[l, h % 2],
                device_id=(lane_nxt(l),),
                device_id_type=pl.DeviceIdType.MESH,
            )
            rdma.start()
            return rdma

        def ag_send(l, h):
            src = stage.at[l] if h == 0 else ag_buf.at[l, (h - 1) % 3]
            rdma = pltpu.make_async_remote_copy(
                src_ref=src,
                dst_ref=ag_buf.at[l, h % 3],
                send_sem=ag_send_sems.at[l, h % 3],
                recv_sem=ag_recv_sems.at[l, h % 3],
                device_id=(lane_nxt(l),),
                device_id_type=pl.DeviceIdType.MESH,
            )
            rdma.start()
            return rdma

        inflight = [None] * LANES
        for l in range(LANES):
            rows = pl.ds(lane_rank(l) * CHUNK, CHUNK)
            stage[l] = out_ref[rows, cols(l)].astype(jnp.bfloat16)
            inflight[l] = rs_send(l, 0)

        for h in range(N_DEV - 1):
            slot = h % 2
            for l in range(LANES):
                inflight[l].wait()
                recv_chunk = (lane_rank(l) - h - 1) % N_DEV
                rows = pl.ds(recv_chunk * CHUNK, CHUNK)
                acc = out_ref[rows, cols(l)] + rs_buf[l, slot].astype(jnp.float32)
                stage[l] = acc.astype(jnp.bfloat16)
                if h < N_DEV - 2:
                    if h + 1 >= 2:
                        pl.semaphore_wait(rs_credits.at[l, (h + 1) % 2], 1)
                    inflight[l] = rs_send(l, h + 1)
                else:
                    inflight[l] = ag_send(l, 0)
                    out_ref[rows, cols(l)] = acc
                if h <= 4:
                    pl.semaphore_signal(
                        rs_credits.at[l, slot], inc=1,
                        device_id=(lane_prv(l),),
                        device_id_type=pl.DeviceIdType.MESH,
                    )

        for h in range(N_DEV - 1):
            slot = h % 3
            for l in range(LANES):
                inflight[l].wait()
                if h < N_DEV - 2:
                    if h + 1 >= 3:
                        pl.semaphore_wait(ag_credits.at[l, (h + 1) % 3], 1)
                    inflight[l] = ag_send(l, h + 1)
                if 1 <= h <= 4:
                    pl.semaphore_signal(
                        ag_credits.at[l, (h - 1) % 3], inc=1,
                        device_id=(lane_prv(l),),
                        device_id_type=pl.DeviceIdType.MESH,
                    )
                recv_chunk = (lane_rank(l) - h) % N_DEV
                rows = pl.ds(recv_chunk * CHUNK, CHUNK)
                out_ref[rows, cols(l)] = ag_buf[l, slot].astype(jnp.float32)

    return pl.pallas_call(
        body,
        out_shape=jax.ShapeDtypeStruct((M, N), jnp.float32),
        in_specs=[
            pl.BlockSpec(memory_space=pltpu.VMEM),
            pl.BlockSpec(memory_space=pltpu.VMEM),
            pl.BlockSpec(memory_space=pltpu.VMEM),
            pl.BlockSpec(memory_space=pltpu.VMEM),
        ],
        out_specs=pl.BlockSpec(memory_space=pltpu.VMEM),
        scratch_shapes=[
            pltpu.VMEM((LANES, CHUNK, LCOL), jnp.bfloat16),
            pltpu.VMEM((LANES, 2, CHUNK, LCOL), jnp.bfloat16),
            pltpu.SemaphoreType.DMA((LANES, 2)),
            pltpu.SemaphoreType.DMA((LANES, 2)),
            pltpu.SemaphoreType.REGULAR((LANES, 2)),
            pltpu.VMEM((LANES, 3, CHUNK, LCOL), jnp.bfloat16),
            pltpu.SemaphoreType.DMA((LANES, 3)),
            pltpu.SemaphoreType.DMA((LANES, 3)),
            pltpu.SemaphoreType.REGULAR((LANES, 3)),
        ],
        compiler_params=pltpu.CompilerParams(
            collective_id=0,
            vmem_limit_bytes=60 * 1024 * 1024,
        ),
    )(x, w_mat, scale_x, scale_w)
